# baseline (speedup 1.0000x reference)
import functools

import jax
import jax.numpy as jnp
from jax.experimental import pallas as pl
from jax.experimental.pallas import tpu as pltpu


def _make_kernel(nf, nd, ft, dt):
    def _body(x_ref, we_ref, be_ref, wd_ref, bdf_ref, bdc_ref, out_ref,
              a_ref, h_ref):
        j = pl.program_id(1)
        nt = (((1,), (1,)), ((), ()))

        @pl.when(j == 0)
        def _():
            a_ref[...] = (x_ref[...] - bdf_ref[...]).astype(jnp.bfloat16)

        @pl.when(j < nf)
        def _():
            pre = jax.lax.dot_general(
                a_ref[...], we_ref[...], nt,
                preferred_element_type=jnp.float32,
            )
            h_ref[j] = jnp.maximum(pre + be_ref[...], 0.0).astype(jnp.bfloat16)

        @pl.when(j >= nf)
        def _():
            acc = jnp.broadcast_to(bdc_ref[0], out_ref.shape).astype(jnp.float32)
            for jj in range(nf):
                acc = acc + jax.lax.dot_general(
                    h_ref[jj], wd_ref[0, :, jj * ft:(jj + 1) * ft], nt,
                    preferred_element_type=jnp.float32,
                )
            out_ref[...] = acc

    return _body


@functools.partial(jax.jit, static_argnames=("mt", "ft", "dt"))
def _fused_sae(x, we, be2, wd3, bdf, bdc, mt, ft, dt):
    m, d = x.shape
    fdim = we.shape[0]
    nf = fdim // ft
    nd = d // dt
    grid = (m // mt, nf + nd)
    body = _make_kernel(nf, nd, ft, dt)
    return pl.pallas_call(
        body,
        grid=grid,
        in_specs=[
            pl.BlockSpec((mt, d), lambda i, j: (i, 0)),
            pl.BlockSpec((ft, d),
                         lambda i, j: (jnp.minimum(j, nf - 1), 0)),
            pl.BlockSpec((1, ft),
                         lambda i, j: (0, jnp.minimum(j, nf - 1))),
            pl.BlockSpec((1, dt, fdim),
                         lambda i, j: (jnp.maximum(j - nf, 0), 0, 0)),
            pl.BlockSpec((1, d), lambda i, j: (0, 0)),
            pl.BlockSpec((1, 1, dt),
                         lambda i, j: (jnp.maximum(j - nf, 0), 0, 0)),
        ],
        out_specs=pl.BlockSpec(
            (mt, dt), lambda i, j: (i, jnp.maximum(j - nf, 0))),
        out_shape=jax.ShapeDtypeStruct((m, d), jnp.float32),
        scratch_shapes=[
            pltpu.VMEM((mt, d), jnp.bfloat16),
            pltpu.VMEM((nf, mt, ft), jnp.bfloat16),
        ],
        compiler_params=pltpu.CompilerParams(
            dimension_semantics=("parallel", "arbitrary"),
            vmem_limit_bytes=64 * 1024 * 1024,
        ),
    )(x, we, be2, wd3, bdf, bdc)


def kernel(x, W_enc, b_enc, W_dec, b_dec):
    m, d = x.shape
    fdim = W_enc.shape[0]
    mt = min(512, m)
    ft = min(2048, fdim)
    dt = min(256, d)
    we = W_enc.astype(jnp.bfloat16)
    wd3 = W_dec.astype(jnp.bfloat16).reshape(d // dt, dt, fdim)
    be2 = b_enc.reshape(1, fdim)
    bdf = b_dec.reshape(1, d)
    bdc = b_dec.reshape(d // dt, 1, dt)
    return _fused_sae(x, we, be2, wd3, bdf, bdc, mt, ft, dt)


# speedup vs baseline: 1.0410x; 1.0410x over previous
import functools

import jax
import jax.numpy as jnp
from jax.experimental import pallas as pl
from jax.experimental.pallas import tpu as pltpu


def _fused_sae_kernel(x_ref, we_ref, be_ref, wd_ref, bd_ref, out_ref, a_ref):
    f = pl.program_id(1)

    @pl.when(f == 0)
    def _():
        a_ref[...] = (x_ref[...] - bd_ref[...]).astype(jnp.bfloat16)

    a = a_ref[...]
    pre = jax.lax.dot_general(
        a, we_ref[...], (((1,), (1,)), ((), ())),
        preferred_element_type=jnp.float32,
    )
    h = jnp.maximum(pre + be_ref[...], 0.0).astype(jnp.bfloat16)
    contrib = jax.lax.dot_general(
        h, wd_ref[...], (((1,), (1,)), ((), ())),
        preferred_element_type=jnp.float32,
    )

    @pl.when(f == 0)
    def _():
        out_ref[...] = contrib + bd_ref[...]

    @pl.when(f != 0)
    def _():
        out_ref[...] += contrib


@functools.partial(jax.jit, static_argnames=("mt", "ft"))
def _fused_sae(x, we, be2, wd, bd2, mt, ft):
    m, d = x.shape
    fdim = we.shape[0]
    grid = (m // mt, fdim // ft)
    return pl.pallas_call(
        _fused_sae_kernel,
        grid=grid,
        in_specs=[
            pl.BlockSpec((mt, d), lambda i, j: (i, 0)),
            pl.BlockSpec((ft, d), lambda i, j: (j, 0)),
            pl.BlockSpec((1, ft), lambda i, j: (0, j)),
            pl.BlockSpec((d, ft), lambda i, j: (0, j)),
            pl.BlockSpec((1, d), lambda i, j: (0, 0)),
        ],
        out_specs=pl.BlockSpec((mt, d), lambda i, j: (i, 0)),
        out_shape=jax.ShapeDtypeStruct((m, d), jnp.float32),
        scratch_shapes=[pltpu.VMEM((mt, d), jnp.bfloat16)],
        compiler_params=pltpu.CompilerParams(
            dimension_semantics=("parallel", "arbitrary"),
            vmem_limit_bytes=64 * 1024 * 1024,
        ),
    )(x, we, be2, wd, bd2)


def kernel(x, W_enc, b_enc, W_dec, b_dec):
    m, d = x.shape
    fdim = W_enc.shape[0]
    mt = min(512, m)
    ft = min(2048, fdim)
    we = W_enc.astype(jnp.bfloat16)
    wd = W_dec.astype(jnp.bfloat16)
    be2 = b_enc.reshape(1, fdim)
    bd2 = b_dec.reshape(1, d)
    return _fused_sae(x, we, be2, wd, bd2, mt, ft)
